# Initial kernel scaffold; baseline (speedup 1.0000x reference)
import jax
import jax.numpy as jnp
from jax.experimental import pallas as pl


def kernel(x_s, x_t, edge_index, edge_attr, u, batch_e, batch_s, batch_t, params):
    raise NotImplementedError("write your pallas kernel here")


# speedup vs baseline: 1.9213x; 1.9213x over previous
import functools
import jax
import jax.numpy as jnp
from jax import lax
from jax.experimental import pallas as pl
from jax.experimental.pallas import tpu as pltpu
from jax.experimental.pallas import tpu_sc as plsc

F_XS = 128
F_XT = 128
F_E = 16
F_U = 16
N_S = 10000
N_T = 10000
E = 160000
B = 16
D1 = F_E + F_XT

NC = 2
NS = 16
NW = NC * NS
CH = 128
NCH = E // CH
CH_IT = (NCH + NW - 1) // NW
ROWS_T = N_S // NS


def _lrelu(x):
    return jnp.where(x >= 0, x, 0.1 * x)


def _make_gather(dims):
    n = len(dims)
    scratch = []
    for d in dims:
        scratch.append(pltpu.VMEM((CH,), jnp.int32))
        scratch.append(pltpu.VMEM((CH, d), jnp.float32))
    scratch.append(pltpu.SemaphoreType.DMA)

    @functools.partial(
        pl.kernel,
        mesh=plsc.VectorSubcoreMesh(core_axis_name="c", subcore_axis_name="s",
                                    num_cores=NC),
        out_type=[jax.ShapeDtypeStruct((E, d), jnp.float32) for d in dims],
        scratch_types=scratch,
        compiler_params=pltpu.CompilerParams(use_tc_tiling_on_sc=False),
    )
    def k(*refs):
        tabs = refs[0:2 * n:2]
        idxs = refs[1:2 * n:2]
        outs = refs[2 * n:3 * n]
        bufs = refs[3 * n:3 * n + 2 * n]
        sem = refs[-1]
        wid = lax.axis_index("s") * NC + lax.axis_index("c")

        def body(j, carry):
            ci = j * NW + wid

            @pl.when(ci < NCH)
            def _():
                base = ci * CH
                for t in range(n):
                    ibuf = bufs[2 * t]
                    rbuf = bufs[2 * t + 1]
                    pltpu.sync_copy(idxs[t].at[pl.ds(base, CH)], ibuf)
                    pltpu.async_copy(tabs[t].at[ibuf], rbuf, sem).wait()
                    pltpu.sync_copy(rbuf, outs[t].at[pl.ds(base, CH)])
            return carry

        lax.fori_loop(0, CH_IT, body, 0)

    return k


def _make_scatter(widths, nseg):
    n = len(widths)
    rows_t = nseg // NS
    uw = sorted(set(widths))
    scratch = []
    for w in uw:
        scratch.append(pltpu.VMEM_SHARED((nseg, w), jnp.float32))
    for w in uw:
        scratch.append(pltpu.VMEM((CH, w), jnp.float32))
    scratch.append(pltpu.VMEM((CH,), jnp.int32))
    scratch.append(pltpu.SemaphoreType.DMA)

    @functools.partial(
        pl.kernel,
        mesh=plsc.VectorSubcoreMesh(core_axis_name="c", subcore_axis_name="s",
                                    num_cores=NC),
        out_type=[jax.ShapeDtypeStruct((NC, nseg, w), jnp.float32)
                  for w in widths],
        scratch_types=scratch,
        compiler_params=pltpu.CompilerParams(use_tc_tiling_on_sc=False),
    )
    def k(*refs):
        vals = refs[0:n]
        idx = refs[n]
        zrefs = {w: refs[n + 1 + i] for i, w in enumerate(uw)}
        outs = refs[n + 1 + len(uw):n + 1 + len(uw) + n]
        accs = {w: refs[n + 1 + len(uw) + n + i] for i, w in enumerate(uw)}
        vbufs = {w: refs[n + 1 + len(uw) + n + len(uw) + i]
                 for i, w in enumerate(uw)}
        ibuf = refs[-2]
        sem = refs[-1]
        del sem
        c = lax.axis_index("c")
        s = lax.axis_index("s")
        wid = s * NC + c
        for p in range(n):
            w = widths[p]
            acc = accs[w]
            pltpu.sync_copy(zrefs[w],
                            acc.at[pl.ds(s * rows_t, rows_t)])
            plsc.subcore_barrier()

            def body(j, carry):
                ci = j * NW + wid

                @pl.when(ci < NCH)
                def _():
                    base = ci * CH
                    pltpu.sync_copy(idx.at[pl.ds(base, CH)], ibuf)
                    pltpu.sync_copy(vals[p].at[pl.ds(base, CH)], vbufs[w])
                    pltpu.sync_copy(vbufs[w], acc.at[ibuf], add=True)
                return carry

            lax.fori_loop(0, CH_IT, body, 0)
            plsc.subcore_barrier()
            pltpu.sync_copy(acc.at[pl.ds(s * rows_t, rows_t)],
                            outs[p].at[c, pl.ds(s * rows_t, rows_t)])
            plsc.subcore_barrier()

    return k


BLK_E = 640
GRID_E = E // BLK_E


def _t1_body(xs_ref, xt_ref, ea_ref, be_ref, u_ref,
             w1xs, w1xt, w1ea, w1ue, b1, w2, b2,
             wsxt, wse, bsa, wsb, bsb,
             e_out, p1_out, p2_out, p3_out, p4_out):
    xs = xs_ref[...]
    xt = xt_ref[...]
    iot = lax.broadcasted_iota(jnp.int32, (1, B), 1).astype(jnp.float32)
    ue = (be_ref[...] == iot).astype(jnp.float32) @ u_ref[...]
    h = (xs @ w1xs[...] + xt @ w1xt[...] + ea_ref[...] @ w1ea[...]
         + ue @ w1ue[...] + b1[...])
    e2 = _lrelu(h) @ w2[...] + b2[...]
    e_out[...] = e2
    m = _lrelu(xt @ wsxt[...] + e2 @ wse[...] + bsa[...]) @ wsb[...] + bsb[...]
    p2 = m * m
    p1_out[...] = m
    p2_out[...] = p2
    p3_out[...] = p2 * m
    p4_out[...] = p2 * p2


def _edge_spec(d):
    return pl.BlockSpec((BLK_E, d), lambda i: (i, 0))


def _full_spec(shape):
    nd = len(shape)
    return pl.BlockSpec(shape, lambda i: (0,) * nd)


BLK_N = 1000
GRID_N = N_S // BLK_N


def _t2_body(mom_ref, cnt_ref, xs_ref, bs_ref, u_ref,
             wxs, wcnt, wmean, wstd, wskew, wkurt, wu, ba, wb, bb,
             out_ref):
    cnt = cnt_ref[0, :, 0:1] + cnt_ref[1, :, 0:1]
    rc = 1.0 / jnp.maximum(cnt, 1.0)
    m1 = (mom_ref[0, 0] + mom_ref[1, 0]) * rc
    m2 = (mom_ref[0, 1] + mom_ref[1, 1]) * rc
    m3 = (mom_ref[0, 2] + mom_ref[1, 2]) * rc
    m4 = (mom_ref[0, 3] + mom_ref[1, 3]) * rc
    var = jnp.maximum(m2 - m1 * m1, 0.0)
    vs = var + 1e-6
    std = jnp.sqrt(vs)
    c3 = m3 - 3.0 * m1 * m2 + 2.0 * m1 * m1 * m1
    c4 = m4 - 4.0 * m1 * m3 + 6.0 * m1 * m1 * m2 - 3.0 * m1 * m1 * m1 * m1
    skew = c3 / (vs * std)
    kurt = c4 / (vs * vs)
    iot = lax.broadcasted_iota(jnp.int32, (1, B), 1).astype(jnp.float32)
    oh = (bs_ref[...] == iot)
    ug = oh.astype(jnp.float32) @ u_ref[...]
    h = (xs_ref[...] @ wxs[...] + cnt * wcnt[...] + m1 @ wmean[...]
         + std @ wstd[...] + skew @ wskew[...] + kurt @ wkurt[...]
         + ug @ wu[...] + ba[...])
    out_ref[...] = _lrelu(h) @ wb[...] + bb[...]


def _t3_body(xsn_ref, e_ref, wx, we, b1, wb, b2, out_ref):
    m = _lrelu(xsn_ref[...] @ wx[...] + e_ref[...] @ we[...] + b1[...])
    out_ref[...] = m @ wb[...] + b2[...]


def _t4_body(xt_ref, agg_ref, bt_ref, bs_ref, xsn_ref, u_ref,
             wxt, wagg, wu, ba, wb, bb,
             out_x, out_ss, out_sc, out_ts, out_tc):
    i = pl.program_id(0)
    agg = agg_ref[0] + agg_ref[1]
    iot = lax.broadcasted_iota(jnp.int32, (1, B), 1).astype(jnp.float32)
    oh_t = (bt_ref[...] == iot).astype(jnp.float32)
    oh_s = (bs_ref[...] == iot).astype(jnp.float32)
    ug = oh_t @ u_ref[...]
    h = xt_ref[...] @ wxt[...] + agg @ wagg[...] + ug @ wu[...] + ba[...]
    xtn = _lrelu(h) @ wb[...] + bb[...]
    out_x[...] = xtn
    dn = (((0,), (0,)), ((), ()))
    ones = jnp.ones((BLK_N, F_XS), jnp.float32)
    ss = lax.dot_general(oh_s, xsn_ref[...], dn)
    ts = lax.dot_general(oh_t, xtn, dn)
    sc_ = lax.dot_general(oh_s, ones, dn)
    tc_ = lax.dot_general(oh_t, ones, dn)

    @pl.when(i == 0)
    def _():
        out_ss[...] = ss
        out_ts[...] = ts
        out_sc[...] = sc_
        out_tc[...] = tc_

    @pl.when(i > 0)
    def _():
        out_ss[...] += ss
        out_ts[...] += ts
        out_sc[...] += sc_
        out_tc[...] += tc_


def _t5_body(u_ref, ss, sc_, ts, tc_, wu, ws, wt, b1, w2, b2, out_ref):
    s_mean = ss[...] / jnp.maximum(sc_[...], 1.0)
    t_mean = ts[...] / jnp.maximum(tc_[...], 1.0)
    h = (u_ref[...] @ wu[...] + s_mean @ ws[...] + t_mean @ wt[...] + b1[...])
    out_ref[...] = _lrelu(h) @ w2[...] + b2[...]


_sc_cache = {}


def _sc(name):
    if name not in _sc_cache:
        _sc_cache['g3'] = _make_gather([F_XS, F_XT])
        _sc_cache['g1'] = _make_gather([F_XS])
        _sc_cache['ss'] = _make_scatter([D1, D1, D1, D1, 16], N_S)
        _sc_cache['st'] = _make_scatter([D1], N_T)
    return _sc_cache[name]


@jax.jit
def kernel(x_s, x_t, edge_index, edge_attr, u, batch_e, batch_s, batch_t,
           params):
    p = params
    src = edge_index[0].astype(jnp.int32)
    tgt = edge_index[1].astype(jnp.int32)
    be = batch_e.astype(jnp.int32)
    bs_f = batch_s.astype(jnp.float32).reshape(N_S, 1)
    bt_f = batch_t.astype(jnp.float32).reshape(N_T, 1)
    ones16 = jnp.ones((E, 16), jnp.float32)
    z144 = jnp.zeros((ROWS_T, D1), jnp.float32)
    z16 = jnp.zeros((ROWS_T, 16), jnp.float32)

    xs_g, xt_g = _sc('g3')(x_s, src, x_t, tgt)
    be_f = be.astype(jnp.float32).reshape(E, 1)

    r1 = lambda b: b.reshape(1, -1)
    e_new, p1, p2, p3, p4 = pl.pallas_call(
        _t1_body,
        grid=(GRID_E,),
        in_specs=[_edge_spec(F_XS), _edge_spec(F_XT), _edge_spec(F_E),
                  _edge_spec(1), _full_spec((B, F_U)),
                  _full_spec((F_XS, F_E)), _full_spec((F_XT, F_E)),
                  _full_spec((F_E, F_E)), _full_spec((F_U, F_E)),
                  _full_spec((1, F_E)), _full_spec((F_E, F_E)),
                  _full_spec((1, F_E)),
                  _full_spec((F_XT, D1)), _full_spec((F_E, D1)),
                  _full_spec((1, D1)), _full_spec((D1, D1)),
                  _full_spec((1, D1))],
        out_specs=[_edge_spec(F_E)] + [_edge_spec(D1)] * 4,
        out_shape=[jax.ShapeDtypeStruct((E, F_E), jnp.float32)]
        + [jax.ShapeDtypeStruct((E, D1), jnp.float32)] * 4,
    )(xs_g, xt_g, edge_attr, be_f, u,
      p['We1'][:F_XS], p['We1'][F_XS:F_XS + F_XT],
      p['We1'][F_XS + F_XT:F_XS + F_XT + F_E], p['We1'][F_XS + F_XT + F_E:],
      r1(p['be1']), p['We2'], r1(p['be2']),
      p['Ws1a'][:F_XT], p['Ws1a'][F_XT:], r1(p['bs1a']),
      p['Ws1b'], r1(p['bs1b']))

    s1, s2, s3, s4, cnt = _sc('ss')(p1, p2, p3, p4, ones16, src,
                                    z16, z144)
    mom = jnp.stack([s1, s2, s3, s4], axis=1)

    wa = p['Ws2a']
    x_s_new = pl.pallas_call(
        _t2_body,
        grid=(GRID_N,),
        in_specs=[pl.BlockSpec((NC, 4, BLK_N, D1), lambda i: (0, 0, i, 0)),
                  pl.BlockSpec((NC, BLK_N, 16), lambda i: (0, i, 0)),
                  pl.BlockSpec((BLK_N, F_XS), lambda i: (i, 0)),
                  pl.BlockSpec((BLK_N, 1), lambda i: (i, 0)),
                  _full_spec((B, F_U)),
                  _full_spec((F_XS, F_XS)), _full_spec((1, F_XS)),
                  _full_spec((D1, F_XS)), _full_spec((D1, F_XS)),
                  _full_spec((D1, F_XS)), _full_spec((D1, F_XS)),
                  _full_spec((F_U, F_XS)), _full_spec((1, F_XS)),
                  _full_spec((F_XS, F_XS)), _full_spec((1, F_XS))],
        out_specs=pl.BlockSpec((BLK_N, F_XS), lambda i: (i, 0)),
        out_shape=jax.ShapeDtypeStruct((N_S, F_XS), jnp.float32),
    )(mom, cnt, x_s, bs_f, u,
      wa[:F_XS], wa[F_XS:F_XS + 1], wa[F_XS + 1:F_XS + 1 + D1],
      wa[F_XS + 1 + D1:F_XS + 1 + 2 * D1],
      wa[F_XS + 1 + 2 * D1:F_XS + 1 + 3 * D1],
      wa[F_XS + 1 + 3 * D1:F_XS + 1 + 4 * D1],
      wa[F_XS + 1 + 4 * D1:], r1(p['bs2a']), p['Ws2b'], r1(p['bs2b']))

    (xsn_g,) = _sc('g1')(x_s_new, src)

    tmsg = pl.pallas_call(
        _t3_body,
        grid=(GRID_E,),
        in_specs=[_edge_spec(F_XS), _edge_spec(F_E),
                  _full_spec((F_XS, D1)), _full_spec((F_E, D1)),
                  _full_spec((1, D1)), _full_spec((D1, D1)),
                  _full_spec((1, D1))],
        out_specs=_edge_spec(D1),
        out_shape=jax.ShapeDtypeStruct((E, D1), jnp.float32),
    )(xsn_g, e_new, p['Wt1a'][:F_XS], p['Wt1a'][F_XS:], r1(p['bt1a']),
      p['Wt1b'], r1(p['bt1b']))

    (agg2,) = _sc('st')(tmsg, tgt, z144)

    wt2 = p['Wt2a']
    x_t_new, ssum, scnt, tsum, tcnt = pl.pallas_call(
        _t4_body,
        grid=(GRID_N,),
        in_specs=[pl.BlockSpec((BLK_N, F_XT), lambda i: (i, 0)),
                  pl.BlockSpec((NC, BLK_N, D1), lambda i: (0, i, 0)),
                  pl.BlockSpec((BLK_N, 1), lambda i: (i, 0)),
                  pl.BlockSpec((BLK_N, 1), lambda i: (i, 0)),
                  pl.BlockSpec((BLK_N, F_XS), lambda i: (i, 0)),
                  _full_spec((B, F_U)),
                  _full_spec((F_XT, F_XT)), _full_spec((D1, F_XT)),
                  _full_spec((F_U, F_XT)), _full_spec((1, F_XT)),
                  _full_spec((F_XT, F_XT)), _full_spec((1, F_XT))],
        out_specs=[pl.BlockSpec((BLK_N, F_XT), lambda i: (i, 0)),
                   _full_spec((B, F_XS)), _full_spec((B, F_XS)),
                   _full_spec((B, F_XT)), _full_spec((B, F_XT))],
        out_shape=[jax.ShapeDtypeStruct((N_T, F_XT), jnp.float32),
                   jax.ShapeDtypeStruct((B, F_XS), jnp.float32),
                   jax.ShapeDtypeStruct((B, F_XS), jnp.float32),
                   jax.ShapeDtypeStruct((B, F_XT), jnp.float32),
                   jax.ShapeDtypeStruct((B, F_XT), jnp.float32)],
    )(x_t, agg2, bt_f, bs_f, x_s_new, u,
      wt2[:F_XT], wt2[F_XT:F_XT + D1], wt2[F_XT + D1:], r1(p['bt2a']),
      p['Wt2b'], r1(p['bt2b']))

    wg = p['Wg1']
    u_new = pl.pallas_call(
        _t5_body,
        grid=(1,),
        in_specs=[_full_spec((B, F_U)),
                  _full_spec((B, F_XS)), _full_spec((B, F_XS)),
                  _full_spec((B, F_XT)), _full_spec((B, F_XT)),
                  _full_spec((F_U, F_U)), _full_spec((F_XS, F_U)),
                  _full_spec((F_XT, F_U)), _full_spec((1, F_U)),
                  _full_spec((F_U, F_U)), _full_spec((1, F_U))],
        out_specs=_full_spec((B, F_U)),
        out_shape=jax.ShapeDtypeStruct((B, F_U), jnp.float32),
    )(u, ssum, scnt, tsum, tcnt,
      wg[:F_U], wg[F_U:F_U + F_XS], wg[F_U + F_XS:], r1(p['bg1']),
      p['Wg2'], r1(p['bg2']))

    return (x_s_new, x_t_new, e_new, u_new)
